# Initial kernel scaffold; baseline (speedup 1.0000x reference)
import functools

import jax
import jax.numpy as jnp
from jax import lax
from jax.experimental import pallas as pl
from jax.experimental.pallas import tpu as pltpu
from jax.experimental.pallas import tpu_sc as plsc

KL_WEIGHT = 0.01
ENTROPY_WEIGHT = 0.1
ALPHA = 0.25


def _dot_nt(a, b):
    return lax.dot_general(a, b, (((1,), (1,)), ((), ())),
                           preferred_element_type=jnp.float32)


def _tc_body(x_ref, eps_ref, W0_ref, b0_ref, W1_ref, b1_ref, W2_ref, b2_ref,
             Wmu_ref, bmu_ref, Wvar_ref, bvar_ref, protos_ref, psq_ref,
             closest_ref, loss_ref, div_ref, soft_acc, acc_smem, *, B, OUT, K):
    i = pl.program_id(0)
    nb = pl.num_programs(0)

    h = jnp.maximum(_dot_nt(x_ref[...], W0_ref[...]) + b0_ref[...], 0.0)
    h = jnp.maximum(_dot_nt(h, W1_ref[...]) + b1_ref[...], 0.0)
    h = jnp.maximum(_dot_nt(h, W2_ref[...]) + b2_ref[...], 0.0)
    logvar = _dot_nt(h, Wvar_ref[...]) + bvar_ref[...]
    mu = _dot_nt(h, Wmu_ref[...]) + bmu_ref[...]
    sample = mu + eps_ref[...] * jnp.exp(0.5 * logvar)

    sn = jnp.sum(sample * sample, axis=1, keepdims=True)
    sp = _dot_nt(sample, protos_ref[...])
    dists = (sn + psq_ref[...]) - 2.0 * sp

    min_d = jnp.min(dists, axis=1, keepdims=True)
    iota = lax.broadcasted_iota(jnp.int32, dists.shape, 1)
    idx = jnp.min(jnp.where(dists == min_d, iota, K), axis=1, keepdims=True)
    closest_ref[...] = idx

    e = jnp.exp(min_d - dists)
    z = jnp.sum(e, axis=1, keepdims=True)
    col = jnp.sum(e / z, axis=0, keepdims=True)

    @pl.when(i == 0)
    def _init():
        soft_acc[...] = jnp.zeros_like(soft_acc)
        acc_smem[0] = 0.0
        acc_smem[1] = 0.0

    soft_acc[...] += col
    acc_smem[0] += jnp.sum(min_d)
    acc_smem[1] += jnp.sum(1.0 + logvar - mu * mu - jnp.exp(logvar))

    @pl.when(i == nb - 1)
    def _fini():
        sd = soft_acc[...] / float(B) + 1e-6
        sd = sd / jnp.sum(sd)
        ent = jnp.sum(-sd * jnp.log(sd))
        vq = (1.0 + ALPHA) * (acc_smem[0] / float(B * OUT)) + ENTROPY_WEIGHT * ent
        div = -0.5 * acc_smem[1] / float(B)
        div_ref[0, 0] = div
        loss_ref[0, 0] = KL_WEIGHT * div + vq


def _tc_call(x, eps, W0, b0, W1, b1, W2, b2, Wmu, bmu, Wvar, bvar, protos, psq):
    B, IN = x.shape
    H = W0.shape[0]
    OUT = Wmu.shape[0]
    K = protos.shape[0]
    BR = 256 if B % 256 == 0 else B
    nb = B // BR

    full = lambda shape: pl.BlockSpec(shape, lambda i: (0,) * len(shape))
    grid_spec = pl.GridSpec(
        grid=(nb,),
        in_specs=[
            pl.BlockSpec((BR, IN), lambda i: (i, 0)),
            pl.BlockSpec((BR, OUT), lambda i: (i, 0)),
            full((H, IN)), full((1, H)),
            full((H, H)), full((1, H)),
            full((OUT, H)), full((1, OUT)),
            full((OUT, OUT)), full((1, OUT)),
            full((OUT, OUT)), full((1, OUT)),
            full((K, OUT)),
            full((1, K)),
        ],
        out_specs=[
            pl.BlockSpec((BR, 1), lambda i: (i, 0)),
            pl.BlockSpec((1, 1), lambda i: (0, 0)),
            pl.BlockSpec((1, 1), lambda i: (0, 0)),
        ],
    )
    return pl.pallas_call(
        functools.partial(_tc_body, B=B, OUT=OUT, K=K),
        grid_spec=grid_spec,
        out_shape=[
            jax.ShapeDtypeStruct((B, 1), jnp.int32),
            jax.ShapeDtypeStruct((1, 1), jnp.float32),
            jax.ShapeDtypeStruct((1, 1), jnp.float32),
        ],
        scratch_shapes=[
            pltpu.VMEM((1, K), jnp.float32),
            pltpu.SMEM((2,), jnp.float32),
        ],
        compiler_params=pltpu.CompilerParams(
            dimension_semantics=("arbitrary",),
        ),
    )(x, eps, W0, b0, W1, b1, W2, b2, Wmu, bmu, Wvar, bvar, protos, psq)


def _sc_gather(protos, closest):
    K, D = protos.shape
    (B,) = closest.shape
    info = plsc.get_sparse_core_info()
    NC, NS = info.num_cores, info.num_subcores
    NW = NC * NS
    b_per_w = B // NW
    CH = min(128, b_per_w)
    nch = b_per_w // CH
    mesh = plsc.VectorSubcoreMesh(core_axis_name="c", subcore_axis_name="s")

    @functools.partial(
        pl.kernel, mesh=mesh,
        out_type=jax.ShapeDtypeStruct((B, D), jnp.float32),
        scratch_types=[
            pltpu.VMEM((CH,), jnp.int32),
            pltpu.VMEM((CH, D), jnp.float32),
            pltpu.SemaphoreType.DMA,
        ],
    )
    def gather_k(table_hbm, idx_hbm, out_hbm, idx_v, rows_v, sem):
        wid = lax.axis_index("s") * NC + lax.axis_index("c")
        base0 = wid * b_per_w
        for j in range(nch):
            base = base0 + j * CH
            pltpu.sync_copy(idx_hbm.at[pl.ds(base, CH)], idx_v)
            pltpu.async_copy(table_hbm.at[idx_v], rows_v, sem).wait()
            pltpu.sync_copy(rows_v, out_hbm.at[pl.ds(base, CH)])

    return gather_k(protos, closest)


def kernel(x, W0, b0, W1, b1, W2, b2, W_mu, b_mu, W_var, b_var, protos, eps):
    B = x.shape[0]
    psq = jnp.sum(protos * protos, axis=1)[None, :]
    closest, loss, div = _tc_call(
        x, eps, W0, b0[None, :], W1, b1[None, :], W2, b2[None, :],
        W_mu, b_mu[None, :], W_var, b_var[None, :], protos, psq)
    quantized_st = _sc_gather(protos, closest.reshape(B))
    return quantized_st, loss[0, 0], div[0, 0]


# speedup vs baseline: 8.5383x; 8.5383x over previous
import functools

import jax
import jax.numpy as jnp
from jax import lax
from jax.experimental import pallas as pl
from jax.experimental.pallas import tpu as pltpu
from jax.experimental.pallas import tpu_sc as plsc

KL_WEIGHT = 0.01
ENTROPY_WEIGHT = 0.1
ALPHA = 0.25


def _dot_nt(a, b):
    return lax.dot_general(a, b, (((1,), (1,)), ((), ())),
                           preferred_element_type=jnp.float32)


def _tc_body(x_ref, eps_ref, W0_ref, b0_ref, W1_ref, b1_ref, W2_ref, b2_ref,
             Wmu_ref, bmu_ref, Wvar_ref, bvar_ref, protos_ref, psq_ref,
             closest_ref, loss_ref, div_ref, soft_acc, acc_smem, *, B, OUT, K):
    i = pl.program_id(0)
    nb = pl.num_programs(0)

    h = jnp.maximum(_dot_nt(x_ref[...], W0_ref[...]) + b0_ref[...], 0.0)
    h = jnp.maximum(_dot_nt(h, W1_ref[...]) + b1_ref[...], 0.0)
    h = jnp.maximum(_dot_nt(h, W2_ref[...]) + b2_ref[...], 0.0)
    logvar = _dot_nt(h, Wvar_ref[...]) + bvar_ref[...]
    mu = _dot_nt(h, Wmu_ref[...]) + bmu_ref[...]
    sample = mu + eps_ref[...] * jnp.exp(0.5 * logvar)

    sn = jnp.sum(sample * sample, axis=1, keepdims=True)
    sp = _dot_nt(sample, protos_ref[...])
    dists = (sn + psq_ref[...]) - 2.0 * sp

    min_d = jnp.min(dists, axis=1, keepdims=True)
    iota = lax.broadcasted_iota(jnp.int32, dists.shape, 1)
    idx = jnp.min(jnp.where(dists == min_d, iota, K), axis=1, keepdims=True)
    closest_ref[...] = idx

    e = jnp.exp(min_d - dists)
    z = jnp.sum(e, axis=1, keepdims=True)
    col = jnp.sum(e / z, axis=0, keepdims=True)

    @pl.when(i == 0)
    def _init():
        soft_acc[...] = jnp.zeros_like(soft_acc)
        acc_smem[0] = 0.0
        acc_smem[1] = 0.0

    soft_acc[...] += col
    acc_smem[0] += jnp.sum(min_d)
    acc_smem[1] += jnp.sum(1.0 + logvar - mu * mu - jnp.exp(logvar))

    @pl.when(i == nb - 1)
    def _fini():
        sd = soft_acc[...] / float(B) + 1e-6
        sd = sd / jnp.sum(sd)
        ent = jnp.sum(-sd * jnp.log(sd))
        vq = (1.0 + ALPHA) * (acc_smem[0] / float(B * OUT)) + ENTROPY_WEIGHT * ent
        div = -0.5 * acc_smem[1] / float(B)
        div_ref[...] = jnp.broadcast_to(div, (1, 1))
        loss_ref[...] = jnp.broadcast_to(KL_WEIGHT * div + vq, (1, 1))


def _tc_call(x, eps, W0, b0, W1, b1, W2, b2, Wmu, bmu, Wvar, bvar, protos, psq):
    B, IN = x.shape
    H = W0.shape[0]
    OUT = Wmu.shape[0]
    K = protos.shape[0]
    BR = 256 if B % 256 == 0 else B
    nb = B // BR

    full = lambda shape: pl.BlockSpec(shape, lambda i: (0,) * len(shape))
    return pl.pallas_call(
        functools.partial(_tc_body, B=B, OUT=OUT, K=K),
        grid=(nb,),
        in_specs=[
            pl.BlockSpec((BR, IN), lambda i: (i, 0)),
            pl.BlockSpec((BR, OUT), lambda i: (i, 0)),
            full((H, IN)), full((1, H)),
            full((H, H)), full((1, H)),
            full((OUT, H)), full((1, OUT)),
            full((OUT, OUT)), full((1, OUT)),
            full((OUT, OUT)), full((1, OUT)),
            full((K, OUT)),
            full((1, K)),
        ],
        out_specs=[
            pl.BlockSpec((BR, 1), lambda i: (i, 0)),
            pl.BlockSpec((1, 1), lambda i: (0, 0)),
            pl.BlockSpec((1, 1), lambda i: (0, 0)),
        ],
        out_shape=[
            jax.ShapeDtypeStruct((B, 1), jnp.int32),
            jax.ShapeDtypeStruct((1, 1), jnp.float32),
            jax.ShapeDtypeStruct((1, 1), jnp.float32),
        ],
        scratch_shapes=[
            pltpu.VMEM((1, K), jnp.float32),
            pltpu.SMEM((2,), jnp.float32),
        ],
        compiler_params=pltpu.CompilerParams(
            dimension_semantics=("arbitrary",),
        ),
    )(x, eps, W0, b0, W1, b1, W2, b2, Wmu, bmu, Wvar, bvar, protos, psq)


def _sc_gather(protos, closest):
    K, D = protos.shape
    (B,) = closest.shape
    info = plsc.get_sparse_core_info()
    NC, NS = info.num_cores, info.num_subcores
    NW = NC * NS
    b_per_w = B // NW
    CH = min(128, b_per_w)
    nch = b_per_w // CH
    mesh = plsc.VectorSubcoreMesh(core_axis_name="c", subcore_axis_name="s")

    @functools.partial(
        pl.kernel, mesh=mesh,
        out_type=jax.ShapeDtypeStruct((B, D), jnp.float32),
        scratch_types=[
            pltpu.VMEM((CH,), jnp.int32),
            pltpu.VMEM((CH, D), jnp.float32),
            pltpu.SemaphoreType.DMA,
        ],
    )
    def gather_k(table_hbm, idx_hbm, out_hbm, idx_v, rows_v, sem):
        wid = lax.axis_index("s") * NC + lax.axis_index("c")
        base0 = wid * b_per_w
        for j in range(nch):
            base = base0 + j * CH
            pltpu.sync_copy(idx_hbm.at[pl.ds(base, CH)], idx_v)
            pltpu.async_copy(table_hbm.at[idx_v], rows_v, sem).wait()
            pltpu.sync_copy(rows_v, out_hbm.at[pl.ds(base, CH)])

    return gather_k(protos, closest)


def kernel(x, W0, b0, W1, b1, W2, b2, W_mu, b_mu, W_var, b_var, protos, eps):
    B = x.shape[0]
    psq = jnp.sum(protos * protos, axis=1)[None, :]
    closest, loss, div = _tc_call(
        x, eps, W0, b0[None, :], W1, b1[None, :], W2, b2[None, :],
        W_mu, b_mu[None, :], W_var, b_var[None, :], protos, psq)
    quantized_st = _sc_gather(protos, closest.reshape(B))
    return quantized_st, loss[0, 0], div[0, 0]


# speedup vs baseline: 9.5094x; 1.1137x over previous
import functools

import jax
import jax.numpy as jnp
from jax import lax
from jax.experimental import pallas as pl
from jax.experimental.pallas import tpu as pltpu
from jax.experimental.pallas import tpu_sc as plsc

KL_WEIGHT = 0.01
ENTROPY_WEIGHT = 0.1
ALPHA = 0.25


def _dot_nt(a, b):
    return lax.dot_general(a, b, (((1,), (1,)), ((), ())),
                           preferred_element_type=jnp.float32)


def _tc_body(x_ref, eps_ref, W0_ref, b0_ref, W1_ref, b1_ref, W2_ref, b2_ref,
             Wmu_ref, bmu_ref, Wvar_ref, bvar_ref, protos_ref, psq_ref,
             closest_ref, loss_ref, div_ref, soft_acc, acc_smem, *, B, OUT, K):
    i = pl.program_id(0)
    nb = pl.num_programs(0)

    h = jnp.maximum(_dot_nt(x_ref[...], W0_ref[...]) + b0_ref[...], 0.0)
    h = jnp.maximum(_dot_nt(h, W1_ref[...]) + b1_ref[...], 0.0)
    h = jnp.maximum(_dot_nt(h, W2_ref[...]) + b2_ref[...], 0.0)
    logvar = _dot_nt(h, Wvar_ref[...]) + bvar_ref[...]
    mu = _dot_nt(h, Wmu_ref[...]) + bmu_ref[...]
    sample = mu + eps_ref[...] * jnp.exp(0.5 * logvar)

    sn = jnp.sum(sample * sample, axis=1, keepdims=True)
    sp = _dot_nt(sample, protos_ref[...])
    dists = (sn + psq_ref[...]) - 2.0 * sp

    min_d = jnp.min(dists, axis=1, keepdims=True)
    iota = lax.broadcasted_iota(jnp.int32, (1, K), 1)
    idx = jnp.min(jnp.where(dists == min_d, iota, K), axis=1, keepdims=True)
    closest_ref[...] = idx

    e = jnp.exp(min_d - dists)
    z = jnp.sum(e, axis=1, keepdims=True)
    col = jnp.sum(e * (1.0 / z), axis=0, keepdims=True)

    @pl.when(i == 0)
    def _init():
        soft_acc[...] = jnp.zeros_like(soft_acc)
        acc_smem[0] = 0.0
        acc_smem[1] = 0.0

    soft_acc[...] += col
    acc_smem[0] += jnp.sum(min_d)
    acc_smem[1] += jnp.sum(1.0 + logvar - mu * mu - jnp.exp(logvar))

    @pl.when(i == nb - 1)
    def _fini():
        sd = soft_acc[...] / float(B) + 1e-6
        sd = sd / jnp.sum(sd)
        ent = jnp.sum(-sd * jnp.log(sd))
        vq = (1.0 + ALPHA) * (acc_smem[0] / float(B * OUT)) + ENTROPY_WEIGHT * ent
        div = -0.5 * acc_smem[1] / float(B)
        div_ref[...] = jnp.broadcast_to(div, (1, 1))
        loss_ref[...] = jnp.broadcast_to(KL_WEIGHT * div + vq, (1, 1))


def _tc_call(x, eps, W0, b0, W1, b1, W2, b2, Wmu, bmu, Wvar, bvar, protos, psq):
    B, IN = x.shape
    H = W0.shape[0]
    OUT = Wmu.shape[0]
    K = protos.shape[0]
    BR = 512 if B % 512 == 0 else B
    nb = B // BR

    full = lambda shape: pl.BlockSpec(shape, lambda i: (0,) * len(shape))
    return pl.pallas_call(
        functools.partial(_tc_body, B=B, OUT=OUT, K=K),
        grid=(nb,),
        in_specs=[
            pl.BlockSpec((BR, IN), lambda i: (i, 0)),
            pl.BlockSpec((BR, OUT), lambda i: (i, 0)),
            full((H, IN)), full((1, H)),
            full((H, H)), full((1, H)),
            full((OUT, H)), full((1, OUT)),
            full((OUT, OUT)), full((1, OUT)),
            full((OUT, OUT)), full((1, OUT)),
            full((K, OUT)),
            full((1, K)),
        ],
        out_specs=[
            pl.BlockSpec((BR, 1), lambda i: (i, 0)),
            pl.BlockSpec((1, 1), lambda i: (0, 0)),
            pl.BlockSpec((1, 1), lambda i: (0, 0)),
        ],
        out_shape=[
            jax.ShapeDtypeStruct((B, 1), jnp.int32),
            jax.ShapeDtypeStruct((1, 1), jnp.float32),
            jax.ShapeDtypeStruct((1, 1), jnp.float32),
        ],
        scratch_shapes=[
            pltpu.VMEM((1, K), jnp.float32),
            pltpu.SMEM((2,), jnp.float32),
        ],
        compiler_params=pltpu.CompilerParams(
            dimension_semantics=("arbitrary",),
        ),
    )(x, eps, W0, b0, W1, b1, W2, b2, Wmu, bmu, Wvar, bvar, protos, psq)


def _sc_gather(protos, closest):
    K, D = protos.shape
    (B,) = closest.shape
    info = plsc.get_sparse_core_info()
    NC, NS = info.num_cores, info.num_subcores
    NW = NC * NS
    b_per_w = B // NW
    CH = min(128, b_per_w)
    nch = b_per_w // CH
    mesh = plsc.VectorSubcoreMesh(core_axis_name="c", subcore_axis_name="s")

    @functools.partial(
        pl.kernel, mesh=mesh,
        out_type=jax.ShapeDtypeStruct((B, D), jnp.float32),
        scratch_types=[
            pltpu.VMEM((CH,), jnp.int32),
            pltpu.VMEM((CH, D), jnp.float32),
            pltpu.SemaphoreType.DMA,
        ],
    )
    def gather_k(table_hbm, idx_hbm, out_hbm, idx_v, rows_v, sem):
        wid = lax.axis_index("s") * NC + lax.axis_index("c")
        base0 = wid * b_per_w
        for j in range(nch):
            base = base0 + j * CH
            pltpu.sync_copy(idx_hbm.at[pl.ds(base, CH)], idx_v)
            pltpu.async_copy(table_hbm.at[idx_v], rows_v, sem).wait()
            pltpu.sync_copy(rows_v, out_hbm.at[pl.ds(base, CH)])

    return gather_k(protos, closest)


def kernel(x, W0, b0, W1, b1, W2, b2, W_mu, b_mu, W_var, b_var, protos, eps):
    B = x.shape[0]
    psq = jnp.sum(protos * protos, axis=1)[None, :]
    closest, loss, div = _tc_call(
        x, eps, W0, b0[None, :], W1, b1[None, :], W2, b2[None, :],
        W_mu, b_mu[None, :], W_var, b_var[None, :], protos, psq)
    quantized_st = _sc_gather(protos, closest.reshape(B))
    return quantized_st, loss[0, 0], div[0, 0]


# speedup vs baseline: 9.5329x; 1.0025x over previous
import functools

import jax
import jax.numpy as jnp
from jax import lax
from jax.experimental import pallas as pl
from jax.experimental.pallas import tpu as pltpu
from jax.experimental.pallas import tpu_sc as plsc

KL_WEIGHT = 0.01
ENTROPY_WEIGHT = 0.1
ALPHA = 0.25


def _dot_nt(a, b):
    return lax.dot_general(a, b, (((1,), (1,)), ((), ())),
                           preferred_element_type=jnp.float32)


def _tc_body(x_ref, eps_ref, W0_ref, b0_ref, W1_ref, b1_ref, W2_ref, b2_ref,
             Wmu_ref, bmu_ref, Wvar_ref, bvar_ref, protos_ref, psq_ref,
             closest_ref, loss_ref, div_ref, soft_acc, acc_smem, *, B, OUT, K):
    i = pl.program_id(0)
    nb = pl.num_programs(0)

    h = jnp.maximum(_dot_nt(x_ref[...], W0_ref[...]) + b0_ref[...], 0.0)
    h = jnp.maximum(_dot_nt(h, W1_ref[...]) + b1_ref[...], 0.0)
    h = jnp.maximum(_dot_nt(h, W2_ref[...]) + b2_ref[...], 0.0)
    logvar = _dot_nt(h, Wvar_ref[...]) + bvar_ref[...]
    mu = _dot_nt(h, Wmu_ref[...]) + bmu_ref[...]
    sample = mu + eps_ref[...] * jnp.exp(0.5 * logvar)

    sn = jnp.sum(sample * sample, axis=1, keepdims=True)
    sp = _dot_nt(sample, protos_ref[...])
    dists = (sn + psq_ref[...]) - 2.0 * sp

    min_d = jnp.min(dists, axis=1, keepdims=True)
    iota = lax.broadcasted_iota(jnp.int32, (1, K), 1)
    idx = jnp.min(jnp.where(dists == min_d, iota, K), axis=1, keepdims=True)
    closest_ref[...] = idx

    e = jnp.exp(min_d - dists)
    z = jnp.sum(e, axis=1, keepdims=True)
    col = jnp.sum(e * (1.0 / z), axis=0, keepdims=True)

    @pl.when(i == 0)
    def _init():
        soft_acc[...] = jnp.zeros_like(soft_acc)
        acc_smem[0] = 0.0
        acc_smem[1] = 0.0

    soft_acc[...] += col
    acc_smem[0] += jnp.sum(min_d)
    acc_smem[1] += jnp.sum(1.0 + logvar - mu * mu - jnp.exp(logvar))

    @pl.when(i == nb - 1)
    def _fini():
        sd = soft_acc[...] / float(B) + 1e-6
        sd = sd / jnp.sum(sd)
        ent = jnp.sum(-sd * jnp.log(sd))
        vq = (1.0 + ALPHA) * (acc_smem[0] / float(B * OUT)) + ENTROPY_WEIGHT * ent
        div = -0.5 * acc_smem[1] / float(B)
        div_ref[...] = jnp.broadcast_to(div, (1, 1))
        loss_ref[...] = jnp.broadcast_to(KL_WEIGHT * div + vq, (1, 1))


def _tc_call(x, eps, W0, b0, W1, b1, W2, b2, Wmu, bmu, Wvar, bvar, protos, psq):
    B, IN = x.shape
    H = W0.shape[0]
    OUT = Wmu.shape[0]
    K = protos.shape[0]
    BR = 1024 if B % 1024 == 0 else B
    nb = B // BR

    full = lambda shape: pl.BlockSpec(shape, lambda i: (0,) * len(shape))
    return pl.pallas_call(
        functools.partial(_tc_body, B=B, OUT=OUT, K=K),
        grid=(nb,),
        in_specs=[
            pl.BlockSpec((BR, IN), lambda i: (i, 0)),
            pl.BlockSpec((BR, OUT), lambda i: (i, 0)),
            full((H, IN)), full((1, H)),
            full((H, H)), full((1, H)),
            full((OUT, H)), full((1, OUT)),
            full((OUT, OUT)), full((1, OUT)),
            full((OUT, OUT)), full((1, OUT)),
            full((K, OUT)),
            full((1, K)),
        ],
        out_specs=[
            pl.BlockSpec((BR, 1), lambda i: (i, 0)),
            pl.BlockSpec((1, 1), lambda i: (0, 0)),
            pl.BlockSpec((1, 1), lambda i: (0, 0)),
        ],
        out_shape=[
            jax.ShapeDtypeStruct((B, 1), jnp.int32),
            jax.ShapeDtypeStruct((1, 1), jnp.float32),
            jax.ShapeDtypeStruct((1, 1), jnp.float32),
        ],
        scratch_shapes=[
            pltpu.VMEM((1, K), jnp.float32),
            pltpu.SMEM((2,), jnp.float32),
        ],
        compiler_params=pltpu.CompilerParams(
            dimension_semantics=("arbitrary",),
        ),
    )(x, eps, W0, b0, W1, b1, W2, b2, Wmu, bmu, Wvar, bvar, protos, psq)


def _sc_gather(protos, closest):
    K, D = protos.shape
    (B,) = closest.shape
    info = plsc.get_sparse_core_info()
    NC, NS = info.num_cores, info.num_subcores
    NW = NC * NS
    b_per_w = B // NW
    CH = min(128, b_per_w)
    nch = b_per_w // CH
    mesh = plsc.VectorSubcoreMesh(core_axis_name="c", subcore_axis_name="s")

    @functools.partial(
        pl.kernel, mesh=mesh,
        out_type=jax.ShapeDtypeStruct((B, D), jnp.float32),
        scratch_types=[
            pltpu.VMEM((CH,), jnp.int32),
            pltpu.VMEM((CH, D), jnp.float32),
            pltpu.SemaphoreType.DMA,
        ],
    )
    def gather_k(table_hbm, idx_hbm, out_hbm, idx_v, rows_v, sem):
        wid = lax.axis_index("s") * NC + lax.axis_index("c")
        base0 = wid * b_per_w
        for j in range(nch):
            base = base0 + j * CH
            pltpu.sync_copy(idx_hbm.at[pl.ds(base, CH)], idx_v)
            pltpu.async_copy(table_hbm.at[idx_v], rows_v, sem).wait()
            pltpu.sync_copy(rows_v, out_hbm.at[pl.ds(base, CH)])

    return gather_k(protos, closest)


def kernel(x, W0, b0, W1, b1, W2, b2, W_mu, b_mu, W_var, b_var, protos, eps):
    B = x.shape[0]
    psq = jnp.sum(protos * protos, axis=1)[None, :]
    closest, loss, div = _tc_call(
        x, eps, W0, b0[None, :], W1, b1[None, :], W2, b2[None, :],
        W_mu, b_mu[None, :], W_var, b_var[None, :], protos, psq)
    quantized_st = _sc_gather(protos, closest.reshape(B))
    return quantized_st, loss[0, 0], div[0, 0]


# speedup vs baseline: 10.0627x; 1.0556x over previous
import functools

import jax
import jax.numpy as jnp
from jax import lax
from jax.experimental import pallas as pl
from jax.experimental.pallas import tpu as pltpu
from jax.experimental.pallas import tpu_sc as plsc

KL_WEIGHT = 0.01
ENTROPY_WEIGHT = 0.1
ALPHA = 0.25


def _dot_nt(a, b):
    return lax.dot_general(a, b, (((1,), (1,)), ((), ())),
                           preferred_element_type=jnp.float32)


def _tc_body(x_ref, eps_ref, W0_ref, b0_ref, W1_ref, b1_ref, W2_ref, b2_ref,
             Wmu_ref, bmu_ref, Wvar_ref, bvar_ref, protos_ref, psq_ref,
             closest_ref, loss_ref, div_ref, soft_acc, acc_smem, *, B, OUT, K):
    i = pl.program_id(0)
    nb = pl.num_programs(0)

    h = jnp.maximum(_dot_nt(x_ref[...], W0_ref[...]) + b0_ref[...], 0.0)
    h = jnp.maximum(_dot_nt(h, W1_ref[...]) + b1_ref[...], 0.0)
    h = jnp.maximum(_dot_nt(h, W2_ref[...]) + b2_ref[...], 0.0)
    logvar = _dot_nt(h, Wvar_ref[...]) + bvar_ref[...]
    mu = _dot_nt(h, Wmu_ref[...]) + bmu_ref[...]
    sample = mu + eps_ref[...] * jnp.exp(0.5 * logvar)

    sn = jnp.sum(sample * sample, axis=1, keepdims=True)
    sp = _dot_nt(sample, protos_ref[...])
    dists = (sn + psq_ref[...]) - 2.0 * sp

    min_d = jnp.min(dists, axis=1, keepdims=True)
    iota = lax.broadcasted_iota(jnp.int32, (1, K), 1).astype(jnp.float32)
    idx = jnp.min(jnp.where(dists == min_d, iota, float(K)), axis=1,
                  keepdims=True)
    closest_ref[...] = idx.astype(jnp.int32)

    e = jnp.exp(min_d - dists)
    z = jnp.sum(e, axis=1, keepdims=True)
    col = jnp.sum(e * (1.0 / z), axis=0, keepdims=True)

    @pl.when(i == 0)
    def _init():
        soft_acc[...] = jnp.zeros_like(soft_acc)
        acc_smem[0] = 0.0
        acc_smem[1] = 0.0

    soft_acc[...] += col
    acc_smem[0] += jnp.sum(min_d)
    acc_smem[1] += jnp.sum(1.0 + logvar - mu * mu - jnp.exp(logvar))

    @pl.when(i == nb - 1)
    def _fini():
        sd = soft_acc[...] / float(B) + 1e-6
        sd = sd / jnp.sum(sd)
        ent = jnp.sum(-sd * jnp.log(sd))
        vq = (1.0 + ALPHA) * (acc_smem[0] / float(B * OUT)) + ENTROPY_WEIGHT * ent
        div = -0.5 * acc_smem[1] / float(B)
        div_ref[...] = jnp.broadcast_to(div, (1, 1))
        loss_ref[...] = jnp.broadcast_to(KL_WEIGHT * div + vq, (1, 1))


def _tc_call(x, eps, W0, b0, W1, b1, W2, b2, Wmu, bmu, Wvar, bvar, protos, psq):
    B, IN = x.shape
    H = W0.shape[0]
    OUT = Wmu.shape[0]
    K = protos.shape[0]
    BR = 1024 if B % 1024 == 0 else B
    nb = B // BR

    full = lambda shape: pl.BlockSpec(shape, lambda i: (0,) * len(shape))
    return pl.pallas_call(
        functools.partial(_tc_body, B=B, OUT=OUT, K=K),
        grid=(nb,),
        in_specs=[
            pl.BlockSpec((BR, IN), lambda i: (i, 0)),
            pl.BlockSpec((BR, OUT), lambda i: (i, 0)),
            full((H, IN)), full((1, H)),
            full((H, H)), full((1, H)),
            full((OUT, H)), full((1, OUT)),
            full((OUT, OUT)), full((1, OUT)),
            full((OUT, OUT)), full((1, OUT)),
            full((K, OUT)),
            full((1, K)),
        ],
        out_specs=[
            pl.BlockSpec((BR, 1), lambda i: (i, 0)),
            pl.BlockSpec((1, 1), lambda i: (0, 0)),
            pl.BlockSpec((1, 1), lambda i: (0, 0)),
        ],
        out_shape=[
            jax.ShapeDtypeStruct((B, 1), jnp.int32),
            jax.ShapeDtypeStruct((1, 1), jnp.float32),
            jax.ShapeDtypeStruct((1, 1), jnp.float32),
        ],
        scratch_shapes=[
            pltpu.VMEM((1, K), jnp.float32),
            pltpu.SMEM((2,), jnp.float32),
        ],
        compiler_params=pltpu.CompilerParams(
            dimension_semantics=("arbitrary",),
        ),
    )(x, eps, W0, b0, W1, b1, W2, b2, Wmu, bmu, Wvar, bvar, protos, psq)


def _sc_gather(protos, closest):
    K, D = protos.shape
    (B,) = closest.shape
    info = plsc.get_sparse_core_info()
    NC, NS = info.num_cores, info.num_subcores
    NW = NC * NS
    b_per_w = B // NW
    CH = min(128, b_per_w)
    nch = b_per_w // CH
    mesh = plsc.VectorSubcoreMesh(core_axis_name="c", subcore_axis_name="s")

    @functools.partial(
        pl.kernel, mesh=mesh,
        out_type=jax.ShapeDtypeStruct((B, D), jnp.float32),
        scratch_types=[
            pltpu.VMEM((CH,), jnp.int32),
            pltpu.VMEM((CH, D), jnp.float32),
            pltpu.SemaphoreType.DMA,
        ],
    )
    def gather_k(table_hbm, idx_hbm, out_hbm, idx_v, rows_v, sem):
        wid = lax.axis_index("s") * NC + lax.axis_index("c")
        base0 = wid * b_per_w
        for j in range(nch):
            base = base0 + j * CH
            pltpu.sync_copy(idx_hbm.at[pl.ds(base, CH)], idx_v)
            pltpu.async_copy(table_hbm.at[idx_v], rows_v, sem).wait()
            pltpu.sync_copy(rows_v, out_hbm.at[pl.ds(base, CH)])

    return gather_k(protos, closest)


def kernel(x, W0, b0, W1, b1, W2, b2, W_mu, b_mu, W_var, b_var, protos, eps):
    B = x.shape[0]
    psq = jnp.sum(protos * protos, axis=1)[None, :]
    closest, loss, div = _tc_call(
        x, eps, W0, b0[None, :], W1, b1[None, :], W2, b2[None, :],
        W_mu, b_mu[None, :], W_var, b_var[None, :], protos, psq)
    quantized_st = _sc_gather(protos, closest.reshape(B))
    return quantized_st, loss[0, 0], div[0, 0]
